# Initial kernel scaffold; baseline (speedup 1.0000x reference)
import jax
import jax.numpy as jnp
from jax.experimental import pallas as pl


def kernel(center_words, target_words, co_occurrences, weightings, embedding_v, embedding_u, v_bias, u_bias):
    raise NotImplementedError("write your pallas kernel here")


# speedup vs baseline: 1.2296x; 1.2296x over previous
import functools

import jax
import jax.numpy as jnp
from jax import lax
from jax.experimental import pallas as pl
from jax.experimental.pallas import tpu as pltpu
from jax.experimental.pallas import tpu_sc as plsc

_VOCAB = 100000
_EMBED = 128
_BATCH = 16384

_NC = 2
_NS = 16
_NW = _NC * _NS
_BPW = _BATCH // _NW
_CHUNK = 128
_NCHUNK = _BPW // _CHUNK
_L = 16


def _glove_body(cidx_hbm, tidx_hbm, cooc_hbm, wt_hbm, ev_hbm, eu_hbm,
                vb_hbm, ub_hbm, out_hbm,
                cidx_v, tidx_v, cooc_v, wt_v, vrows, urows, vb_v, ub_v,
                prod_v, outvec_v, sem):
    wid = lax.axis_index("s") * _NC + lax.axis_index("c")
    base = wid * _BPW

    pltpu.sync_copy(cooc_hbm.at[pl.ds(base, _BPW)], cooc_v)
    pltpu.sync_copy(wt_hbm.at[pl.ds(base, _BPW)], wt_v)
    pltpu.sync_copy(cidx_hbm.at[pl.ds(base, _BPW)], cidx_v)
    pltpu.sync_copy(tidx_hbm.at[pl.ds(base, _BPW)], tidx_v)

    lacc = jnp.zeros((_L,), jnp.float32)
    for c in range(_NCHUNK):
        off = c * _CHUNK
        cid = cidx_v.at[pl.ds(off, _CHUNK)]
        tid = tidx_v.at[pl.ds(off, _CHUNK)]
        d1 = pltpu.async_copy(ev_hbm.at[cid], vrows, sem)
        d2 = pltpu.async_copy(eu_hbm.at[tid], urows, sem)
        d3 = pltpu.async_copy(vb_hbm.at[cid], vb_v, sem)
        d4 = pltpu.async_copy(ub_hbm.at[tid], ub_v, sem)
        d1.wait()
        d2.wait()
        d3.wait()
        d4.wait()

        def row(r, carry):
            a = vrows[r, pl.ds(0, _L)] * urows[r, pl.ds(0, _L)]
            for j in range(1, _EMBED // _L):
                a = a + vrows[r, pl.ds(j * _L, _L)] * urows[r, pl.ds(j * _L, _L)]
            prod_v[r, pl.ds(0, _L)] = a
            return carry

        lax.fori_loop(0, _CHUNK, row, 0)

        lane = lax.iota(jnp.int32, _L)

        def group(g, acc, _off=off):
            ridx = g * _L + lane
            dvec = plsc.load_gather(prod_v, [ridx, jnp.zeros((_L,), jnp.int32)])
            for j in range(1, _L):
                dvec = dvec + plsc.load_gather(
                    prod_v, [ridx, jnp.full((_L,), j, jnp.int32)])
            cb = vb_v[pl.ds(g * _L, _L)]
            tb = ub_v[pl.ds(g * _L, _L)]
            co = cooc_v[pl.ds(_off + g * _L, _L)]
            w = wt_v[pl.ds(_off + g * _L, _L)]
            err = dvec + cb + tb - co
            return acc + w * err * err

        lacc = lax.fori_loop(0, _CHUNK // _L, group, lacc)

    outvec_v[...] = lacc
    pltpu.sync_copy(outvec_v, out_hbm.at[wid])


@jax.jit
def _glove(cidx, tidx, cooc, wt, ev, eu, vb, ub):
    mesh = plsc.VectorSubcoreMesh(core_axis_name="c", subcore_axis_name="s",
                                  num_cores=_NC, num_subcores=_NS)
    run = pl.kernel(
        _glove_body,
        out_type=jax.ShapeDtypeStruct((_NW, _L), jnp.float32),
        mesh=mesh,
        compiler_params=pltpu.CompilerParams(needs_layout_passes=False),
        scratch_types=[
            pltpu.VMEM((_BPW,), jnp.int32),
            pltpu.VMEM((_BPW,), jnp.int32),
            pltpu.VMEM((_BPW,), jnp.float32),
            pltpu.VMEM((_BPW,), jnp.float32),
            pltpu.VMEM((_CHUNK, _EMBED), jnp.float32),
            pltpu.VMEM((_CHUNK, _EMBED), jnp.float32),
            pltpu.VMEM((_CHUNK,), jnp.float32),
            pltpu.VMEM((_CHUNK,), jnp.float32),
            pltpu.VMEM((_CHUNK, _L), jnp.float32),
            pltpu.VMEM((_L,), jnp.float32),
            pltpu.SemaphoreType.DMA,
        ],
    )
    partials = run(cidx, tidx, cooc, wt, ev, eu, vb, ub)
    return jnp.sum(partials) * jnp.float32(1.0 / _BATCH)


def kernel(center_words, target_words, co_occurrences, weightings,
           embedding_v, embedding_u, v_bias, u_bias):
    cidx = center_words.astype(jnp.int32)
    tidx = target_words.astype(jnp.int32)
    vb = jnp.reshape(v_bias, (_VOCAB,))
    ub = jnp.reshape(u_bias, (_VOCAB,))
    return _glove(cidx, tidx, co_occurrences, weightings,
                  embedding_v, embedding_u, vb, ub)


# speedup vs baseline: 1.3305x; 1.0821x over previous
import functools

import jax
import jax.numpy as jnp
from jax import lax
from jax.experimental import pallas as pl
from jax.experimental.pallas import tpu as pltpu
from jax.experimental.pallas import tpu_sc as plsc

_VOCAB = 100000
_EMBED = 128
_BATCH = 16384

_NC = 2
_NS = 16
_NW = _NC * _NS
_BPW = _BATCH // _NW
_CHUNK = 128
_NCHUNK = _BPW // _CHUNK
_L = 16


def _glove_body(cidx_hbm, tidx_hbm, cooc_hbm, wt_hbm, ev_hbm, eu_hbm,
                vb_hbm, ub_hbm, out_hbm,
                cidx_v, tidx_v, cooc_v, wt_v,
                vrows0, urows0, vb0, ub0,
                vrows1, urows1, vb1, ub1,
                prod_v, outvec_v, sem0, sem1):
    wid = lax.axis_index("s") * _NC + lax.axis_index("c")
    base = wid * _BPW

    pltpu.sync_copy(cooc_hbm.at[pl.ds(base, _BPW)], cooc_v)
    pltpu.sync_copy(wt_hbm.at[pl.ds(base, _BPW)], wt_v)
    pltpu.sync_copy(cidx_hbm.at[pl.ds(base, _BPW)], cidx_v)
    pltpu.sync_copy(tidx_hbm.at[pl.ds(base, _BPW)], tidx_v)

    bufs = ((vrows0, urows0, vb0, ub0, sem0),
            (vrows1, urows1, vb1, ub1, sem1))

    def issue(c):
        vr, ur, vb, ub, sem = bufs[c % 2]
        cid = cidx_v.at[pl.ds(c * _CHUNK, _CHUNK)]
        tid = tidx_v.at[pl.ds(c * _CHUNK, _CHUNK)]
        return (pltpu.async_copy(ev_hbm.at[cid], vr, sem),
                pltpu.async_copy(eu_hbm.at[tid], ur, sem),
                pltpu.async_copy(vb_hbm.at[cid], vb, sem),
                pltpu.async_copy(ub_hbm.at[tid], ub, sem))

    pend = issue(0)
    lane = lax.iota(jnp.int32, _L)
    lacc = jnp.zeros((_L,), jnp.float32)

    for c in range(_NCHUNK):
        for h in pend:
            h.wait()
        if c + 1 < _NCHUNK:
            pend = issue(c + 1)
        vr, ur, vb, ub, _ = bufs[c % 2]
        off = c * _CHUNK

        def row(r, carry, _vr=vr, _ur=ur):
            a = _vr[r, pl.ds(0, _L)] * _ur[r, pl.ds(0, _L)]
            for j in range(1, _EMBED // _L):
                a = a + _vr[r, pl.ds(j * _L, _L)] * _ur[r, pl.ds(j * _L, _L)]
            prod_v[r, pl.ds(0, _L)] = a
            return carry

        lax.fori_loop(0, _CHUNK, row, 0, unroll=4)

        def group(g, acc, _off=off, _vb=vb, _ub=ub):
            ridx = g * _L + lane
            dvec = plsc.load_gather(prod_v, [ridx, jnp.zeros((_L,), jnp.int32)])
            for j in range(1, _L):
                dvec = dvec + plsc.load_gather(
                    prod_v, [ridx, jnp.full((_L,), j, jnp.int32)])
            cb = _vb[pl.ds(g * _L, _L)]
            tb = _ub[pl.ds(g * _L, _L)]
            co = cooc_v[pl.ds(_off + g * _L, _L)]
            w = wt_v[pl.ds(_off + g * _L, _L)]
            err = dvec + cb + tb - co
            return acc + w * err * err

        lacc = lax.fori_loop(0, _CHUNK // _L, group, lacc, unroll=2)

    outvec_v[...] = lacc
    pltpu.sync_copy(outvec_v, out_hbm.at[wid])


@jax.jit
def _glove(cidx, tidx, cooc, wt, ev, eu, vb, ub):
    mesh = plsc.VectorSubcoreMesh(core_axis_name="c", subcore_axis_name="s",
                                  num_cores=_NC, num_subcores=_NS)
    run = pl.kernel(
        _glove_body,
        out_type=jax.ShapeDtypeStruct((_NW, _L), jnp.float32),
        mesh=mesh,
        compiler_params=pltpu.CompilerParams(needs_layout_passes=False),
        scratch_types=[
            pltpu.VMEM((_BPW,), jnp.int32),
            pltpu.VMEM((_BPW,), jnp.int32),
            pltpu.VMEM((_BPW,), jnp.float32),
            pltpu.VMEM((_BPW,), jnp.float32),
            pltpu.VMEM((_CHUNK, _EMBED), jnp.float32),
            pltpu.VMEM((_CHUNK, _EMBED), jnp.float32),
            pltpu.VMEM((_CHUNK,), jnp.float32),
            pltpu.VMEM((_CHUNK,), jnp.float32),
            pltpu.VMEM((_CHUNK, _EMBED), jnp.float32),
            pltpu.VMEM((_CHUNK, _EMBED), jnp.float32),
            pltpu.VMEM((_CHUNK,), jnp.float32),
            pltpu.VMEM((_CHUNK,), jnp.float32),
            pltpu.VMEM((_CHUNK, _L), jnp.float32),
            pltpu.VMEM((_L,), jnp.float32),
            pltpu.SemaphoreType.DMA,
            pltpu.SemaphoreType.DMA,
        ],
    )
    partials = run(cidx, tidx, cooc, wt, ev, eu, vb, ub)
    return jnp.sum(partials) * jnp.float32(1.0 / _BATCH)


def kernel(center_words, target_words, co_occurrences, weightings,
           embedding_v, embedding_u, v_bias, u_bias):
    cidx = center_words.astype(jnp.int32)
    tidx = target_words.astype(jnp.int32)
    vb = jnp.reshape(v_bias, (_VOCAB,))
    ub = jnp.reshape(u_bias, (_VOCAB,))
    return _glove(cidx, tidx, co_occurrences, weightings,
                  embedding_v, embedding_u, vb, ub)


# speedup vs baseline: 1.3987x; 1.0512x over previous
import functools

import jax
import jax.numpy as jnp
from jax import lax
from jax.experimental import pallas as pl
from jax.experimental.pallas import tpu as pltpu
from jax.experimental.pallas import tpu_sc as plsc

_VOCAB = 100000
_EMBED = 128
_BATCH = 16384

_NC = 2
_NS = 16
_NW = _NC * _NS
_BPW = _BATCH // _NW
_CHUNK = 128
_NCHUNK = _BPW // _CHUNK
_L = 16


def _glove_body(cidx_hbm, tidx_hbm, cooc_hbm, wt_hbm, ev_hbm, eu_hbm,
                vb_hbm, ub_hbm, out_hbm,
                cidx_v, tidx_v, cooc_v, wt_v,
                vrows0, urows0, vb0, ub0,
                vrows1, urows1, vb1, ub1,
                prod_v, outvec_v, sem0, sem1):
    wid = lax.axis_index("s") * _NC + lax.axis_index("c")
    base = wid * _BPW

    pltpu.sync_copy(cooc_hbm.at[pl.ds(base, _BPW)], cooc_v)
    pltpu.sync_copy(wt_hbm.at[pl.ds(base, _BPW)], wt_v)
    pltpu.sync_copy(cidx_hbm.at[pl.ds(base, _BPW)], cidx_v)
    pltpu.sync_copy(tidx_hbm.at[pl.ds(base, _BPW)], tidx_v)

    bufs = ((vrows0, urows0, vb0, ub0, sem0),
            (vrows1, urows1, vb1, ub1, sem1))

    def issue(c):
        vr, ur, vb, ub, sem = bufs[c % 2]
        cid = cidx_v.at[pl.ds(c * _CHUNK, _CHUNK)]
        tid = tidx_v.at[pl.ds(c * _CHUNK, _CHUNK)]
        return (pltpu.async_copy(ev_hbm.at[cid], vr, sem),
                pltpu.async_copy(eu_hbm.at[tid], ur, sem),
                pltpu.async_copy(vb_hbm.at[cid], vb, sem),
                pltpu.async_copy(ub_hbm.at[tid], ub, sem))

    pend = issue(0)
    lane = lax.iota(jnp.int32, _L)
    lacc = jnp.zeros((_L,), jnp.float32)

    for c in range(_NCHUNK):
        for h in pend:
            h.wait()
        if c + 1 < _NCHUNK:
            pend = issue(c + 1)
        vr, ur, vb, ub, _ = bufs[c % 2]
        off = c * _CHUNK

        @plsc.parallel_loop(0, _CHUNK, unroll=4)
        def _row(r, _vr=vr, _ur=ur):
            a = _vr[r, pl.ds(0, _L)] * _ur[r, pl.ds(0, _L)]
            for j in range(1, _EMBED // _L):
                a = a + _vr[r, pl.ds(j * _L, _L)] * _ur[r, pl.ds(j * _L, _L)]
            prod_v[r, pl.ds(0, _L)] = a

        @plsc.parallel_loop(0, _CHUNK // _L, unroll=2, carry=lacc)
        def _group(g, acc, _off=off, _vb=vb, _ub=ub):
            ridx = g * _L + lane
            dvec = plsc.load_gather(prod_v, [ridx, jnp.zeros((_L,), jnp.int32)])
            for j in range(1, _L):
                dvec = dvec + plsc.load_gather(
                    prod_v, [ridx, jnp.full((_L,), j, jnp.int32)])
            cb = _vb[pl.ds(g * _L, _L)]
            tb = _ub[pl.ds(g * _L, _L)]
            co = cooc_v[pl.ds(_off + g * _L, _L)]
            w = wt_v[pl.ds(_off + g * _L, _L)]
            err = dvec + cb + tb - co
            return acc + w * err * err

        lacc = _group

    outvec_v[...] = lacc
    pltpu.sync_copy(outvec_v, out_hbm.at[wid])


@jax.jit
def _glove(cidx, tidx, cooc, wt, ev, eu, vb, ub):
    mesh = plsc.VectorSubcoreMesh(core_axis_name="c", subcore_axis_name="s",
                                  num_cores=_NC, num_subcores=_NS)
    run = pl.kernel(
        _glove_body,
        out_type=jax.ShapeDtypeStruct((_NW, _L), jnp.float32),
        mesh=mesh,
        compiler_params=pltpu.CompilerParams(needs_layout_passes=False),
        scratch_types=[
            pltpu.VMEM((_BPW,), jnp.int32),
            pltpu.VMEM((_BPW,), jnp.int32),
            pltpu.VMEM((_BPW,), jnp.float32),
            pltpu.VMEM((_BPW,), jnp.float32),
            pltpu.VMEM((_CHUNK, _EMBED), jnp.float32),
            pltpu.VMEM((_CHUNK, _EMBED), jnp.float32),
            pltpu.VMEM((_CHUNK,), jnp.float32),
            pltpu.VMEM((_CHUNK,), jnp.float32),
            pltpu.VMEM((_CHUNK, _EMBED), jnp.float32),
            pltpu.VMEM((_CHUNK, _EMBED), jnp.float32),
            pltpu.VMEM((_CHUNK,), jnp.float32),
            pltpu.VMEM((_CHUNK,), jnp.float32),
            pltpu.VMEM((_CHUNK, _L), jnp.float32),
            pltpu.VMEM((_L,), jnp.float32),
            pltpu.SemaphoreType.DMA,
            pltpu.SemaphoreType.DMA,
        ],
    )
    partials = run(cidx, tidx, cooc, wt, ev, eu, vb, ub)
    return jnp.sum(partials) * jnp.float32(1.0 / _BATCH)


def kernel(center_words, target_words, co_occurrences, weightings,
           embedding_v, embedding_u, v_bias, u_bias):
    cidx = center_words.astype(jnp.int32)
    tidx = target_words.astype(jnp.int32)
    vb = jnp.reshape(v_bias, (_VOCAB,))
    ub = jnp.reshape(u_bias, (_VOCAB,))
    return _glove(cidx, tidx, co_occurrences, weightings,
                  embedding_v, embedding_u, vb, ub)
